# Initial kernel scaffold; baseline (speedup 1.0000x reference)
import jax
import jax.numpy as jnp
from jax.experimental import pallas as pl


def kernel(content, mask, condition):
    raise NotImplementedError("write your pallas kernel here")


# speedup vs baseline: 129.0740x; 129.0740x over previous
import jax
import jax.numpy as jnp
from jax.experimental import pallas as pl
from jax.experimental.pallas import tpu as pltpu

_H = 228
_L = 56
_CCH = 4
_NST = 32 // _CCH


def _mm(a, b):
    return jax.lax.dot_general(
        a, b, (((1,), (0,)), ((), ())),
        precision=jax.lax.Precision.HIGHEST,
        preferred_element_type=jnp.float32)


def _selmat():
    r = jax.lax.broadcasted_iota(jnp.int32, (_L, 226), 1)
    lh = jax.lax.broadcasted_iota(jnp.int32, (_L, 226), 0)
    d = r - 4 * lh
    return ((d == 0) | (d == 2) | (d == 4)).astype(jnp.float32)


def _foldmat():
    h = jax.lax.broadcasted_iota(jnp.int32, (_H, _L), 0)
    lh = jax.lax.broadcasted_iota(jnp.int32, (_H, _L), 1)
    d = h - 4 * lh
    return ((d >= 0) & (d <= 4)).astype(jnp.float32)


def _boxsum(R, A):
    return _mm(_mm(A, R), A.T)


def _expand(M, P):
    return _mm(_mm(P, M), P.T)


def _sim_body(content_ref, mask_ref, cond_ref, simi_ref, wf_ref, acc_ref):
    i = pl.program_id(0)

    @pl.when(i == 0)
    def _():
        acc_ref[...] = jnp.zeros_like(acc_ref)

    c0 = content_ref[...]
    m0 = mask_ref[...]
    cd = cond_ref[...]
    mp = (m0 > 0).astype(jnp.float32)
    cm = c0 * mp
    c2 = cd * cd

    acc_ref[18] += jnp.sum(c0 * c0, axis=0)[:226, :226]
    for n in range(9):
        sh, sw = n // 3, n % 3
        cs = cd[:, sh:sh + 226, sw:sw + 226]
        acc_ref[n] += jnp.sum(cm[:, :226, :226] * cs, axis=0)
        acc_ref[9 + n] += jnp.sum(
            mp[:, :226, :226] * c2[:, sh:sh + 226, sw:sw + 226], axis=0)

    @pl.when(i == _NST - 1)
    def _():
        A = _selmat()
        P = _foldmat()
        eps = jnp.float32(1e-8)
        V = _boxsum(acc_ref[18], A)
        sqV = jnp.sqrt(V) + eps
        nvv = V / (sqV * sqV)
        best_val = jnp.full((_L, _L), -jnp.inf, jnp.float32)
        best_idx = jnp.zeros((_L, _L), jnp.int32)
        for n in range(9):
            D = _boxsum(acc_ref[n], A)
            U = _boxsum(acc_ref[9 + n], A)
            sqU = jnp.sqrt(U) + eps
            nuu = U / (sqU * sqU)
            nvu = D / (sqV * sqU)
            eud = jnp.sqrt(jnp.maximum(nvv + nuu - 2.0 * nvu, 0.0))
            sim = (2.0 - eud) * 0.5
            upd = sim > best_val
            best_val = jnp.where(upd, sim, best_val)
            best_idx = jnp.where(upd, n, best_idx)

        maxval = best_val
        maxidx = jnp.where(maxval > 0, best_idx, 0)

        idx_f = maxidx.astype(jnp.float32)
        pieces = []
        for i2 in range(8):
            idx_s = idx_f[:, i2 * 7:(i2 + 1) * 7]
            sa = idx_f[:, i2 * 4:min((i2 + 1) * 7, _L)]
            ssum = jnp.sum(sa)
            scnt = jnp.sum(sa > 0.0).astype(jnp.float32)
            smean = jnp.round(ssum / (scnt + jnp.float32(1e-8)))
            pieces.append(jnp.where(idx_s > 0.0, idx_s, smean))
        fidx = jnp.concatenate(pieces, axis=1).astype(jnp.int32)

        cnt = jnp.maximum(_expand(jnp.ones((_L, _L), jnp.float32), P), 1.0)
        invc = 1.0 / cnt
        for n in range(9):
            wf_ref[n] = _expand((fidx == n).astype(jnp.float32), P) * invc
        simi_ref[...] = _expand(maxval, P) * invc


def _fold_body(wf_ref, cond_ref, mapped_ref):
    cd = cond_ref[...]
    acc = jnp.zeros((_CCH, _H, _H), jnp.float32)
    for n in range(9):
        sh, sw = n // 3, n % 3
        cs = jnp.pad(cd[:, sh:, sw:], ((0, 0), (0, sh), (0, sw)))
        acc = acc + wf_ref[n][None] * cs
    mapped_ref[...] = acc


@jax.jit
def kernel(content, mask, condition):
    c = content[0]
    m = mask[0]
    cd = condition[0]
    simi, wf = pl.pallas_call(
        _sim_body,
        grid=(_NST,),
        in_specs=[
            pl.BlockSpec((_CCH, _H, _H), lambda i: (i, 0, 0)),
            pl.BlockSpec((_CCH, _H, _H), lambda i: (i, 0, 0)),
            pl.BlockSpec((_CCH, _H, _H), lambda i: (i, 0, 0)),
        ],
        out_specs=[
            pl.BlockSpec((_H, _H), lambda i: (0, 0)),
            pl.BlockSpec((9, _H, _H), lambda i: (0, 0, 0)),
        ],
        out_shape=[
            jax.ShapeDtypeStruct((_H, _H), jnp.float32),
            jax.ShapeDtypeStruct((9, _H, _H), jnp.float32),
        ],
        scratch_shapes=[pltpu.VMEM((19, 226, 226), jnp.float32)],
    )(c, m, cd)

    mapped = pl.pallas_call(
        _fold_body,
        grid=(_NST,),
        in_specs=[
            pl.BlockSpec((9, _H, _H), lambda i: (0, 0, 0)),
            pl.BlockSpec((_CCH, _H, _H), lambda i: (i, 0, 0)),
        ],
        out_specs=pl.BlockSpec((_CCH, _H, _H), lambda i: (i, 0, 0)),
        out_shape=jax.ShapeDtypeStruct((32, _H, _H), jnp.float32),
    )(wf, cd)

    simi_full = jnp.broadcast_to(simi[None, None], (1, 32, _H, _H))
    return mapped[None], simi_full
